# baseline (speedup 1.0000x reference)
import math

import jax
import jax.numpy as jnp
from jax import lax
from jax.experimental import pallas as pl
from jax.experimental.pallas import tpu as pltpu
from jax.experimental.pallas import tpu_sc as plsc

_B, _S, _D = 4, 2048, 1024
_SCALE = math.sqrt(_D)
_NW = 32
_PPW = _S // _NW
_RPW = _B * _PPW
_CH = 8
_NCH = _RPW // _CH
_QPB = _PPW // _CH
_LANES = 16
_VPR = _D // _LANES
_NTOK = 4
_PHALF = _PPW // 2


def _embed_kernel(x_hbm, tok_hbm, pos_hbm, out_hbm, idx_v, pos_v,
                  tok0, tok1, tok2, tok3,
                  gs0, gs1, gs2, gs3, ss0, ss1, ss2, ss3,
                  ph0, ph1, is0, is1, is2, is3):
    toks = (tok0, tok1, tok2, tok3)
    gsems = (gs0, gs1, gs2, gs3)
    ssems = (ss0, ss1, ss2, ss3)
    psems = (ph0, ph1)
    isems = (is0, is1, is2, is3)

    c = lax.axis_index("c")
    s = lax.axis_index("s")
    wid = s * 2 + c
    pos0 = wid * _PPW

    phs = [pltpu.async_copy(pos_hbm.at[pl.ds(pos0 + h * _PHALF, _PHALF)],
                            pos_v.at[pl.ds(h * _PHALF, _PHALF)], psems[h])
           for h in range(2)]
    pwaited = [False, False]

    icopies = [pltpu.async_copy(x_hbm.at[pl.ds(b * _S + pos0, _PPW)],
                                idx_v.at[b], isems[b])
               for b in range(_B)]
    iwaited = [False] * _B

    def start_gather(ch):
        b, q = ch // _QPB, ch % _QPB
        if not iwaited[b]:
            icopies[b].wait()
            iwaited[b] = True
        return pltpu.async_copy(
            tok_hbm.at[idx_v.at[b, pl.ds(q * _CH, _CH)]],
            toks[ch % _NTOK], gsems[ch % _NTOK])

    gathers = [None] * _NCH
    stores = [None] * _NCH
    gathers[0] = start_gather(0)
    gathers[1] = start_gather(1)
    for ch in range(_NCH):
        tb = ch % _NTOK
        b, q = ch // _QPB, ch % _QPB
        if ch + 2 < _NCH:
            if ch >= 2 and stores[ch - 2] is not None:
                stores[ch - 2].wait()
            gathers[ch + 2] = start_gather(ch + 2)
        gathers[ch].wait()
        half = (q * _CH) // _PHALF
        if not pwaited[half]:
            phs[half].wait()
            pwaited[half] = True

        q8 = q * _CH

        def row_body(r, carry):
            def half_body(h, carry2):
                hoff = h * (_VPR // 2 * _LANES)
                for k in range(_VPR // 2):
                    off = hoff + k * _LANES
                    t = toks[tb][r, pl.ds(off, _LANES)]
                    pv = pos_v[q8 + r, pl.ds(off, _LANES)]
                    toks[tb][r, pl.ds(off, _LANES)] = t * _SCALE + pv
                return carry2
            return lax.fori_loop(0, 2, half_body, carry)
        lax.fori_loop(0, _CH, row_body, 0, unroll=False)

        out_base = b * _S + pos0 + q8
        stores[ch] = pltpu.async_copy(
            toks[tb], out_hbm.at[pl.ds(out_base, _CH)], ssems[tb])
    stores[_NCH - 2].wait()
    stores[_NCH - 1].wait()


def kernel(x, token_table, pos_table):
    xf = x.reshape(_B * _S).astype(jnp.int32)
    mesh = plsc.VectorSubcoreMesh(core_axis_name="c", subcore_axis_name="s")
    run = pl.kernel(
        _embed_kernel,
        out_type=jax.ShapeDtypeStruct((_B * _S, _D), jnp.float32),
        mesh=mesh,
        scratch_types=(
            [pltpu.VMEM((_B, _PPW), jnp.int32),
             pltpu.VMEM((_PPW, _D), jnp.float32)]
            + [pltpu.VMEM((_CH, _D), jnp.float32) for _ in range(_NTOK)]
            + [pltpu.SemaphoreType.DMA for _ in range(2 * _NTOK + 6)]
        ),
    )
    out = run(xf, token_table, pos_table)
    return out.reshape(_B, _S, _D)


# speedup vs baseline: 2.1189x; 2.1189x over previous
import math

import jax
import jax.numpy as jnp
from jax import lax
from jax.experimental import pallas as pl
from jax.experimental.pallas import tpu as pltpu
from jax.experimental.pallas import tpu_sc as plsc

_B, _S, _D = 4, 2048, 1024
_SCALE = math.sqrt(_D)
_NW = 32
_NPAIR = 2
_PPW = (_B * _S) // (_NW * _NPAIR)
_RPW = _NPAIR * _PPW
_CH = 16
_NQ = _PPW // _CH
_NCH = _NPAIR * _NQ
_LANES = 16
_VPR = _D // _LANES
_NTOK = 4
_NPOS = 2


def _embed_kernel(x_hbm, tok_hbm, pos_hbm, out_hbm, idx_v,
                  tok0, tok1, tok2, tok3, pos0, pos1,
                  gs0, gs1, gs2, gs3, ps0, ps1, ss0, ss1, ss2, ss3,
                  is0, is1):
    toks = (tok0, tok1, tok2, tok3)
    poss = (pos0, pos1)
    gsems = (gs0, gs1, gs2, gs3)
    psems = (ps0, ps1)
    ssems = (ss0, ss1, ss2, ss3)
    isems = (is0, is1)

    c = lax.axis_index("c")
    s = lax.axis_index("s")
    wid = s * 2 + c
    g = wid // 16
    p0 = lax.rem(wid, 16) * _PPW

    icopies = [pltpu.async_copy(
        x_hbm.at[pl.ds((2 * g + t) * _S + p0, _PPW)], idx_v.at[t], isems[t])
        for t in range(_NPAIR)]
    iwaited = [False] * _NPAIR

    def start_gather(ch):
        q, t = ch // _NPAIR, ch % _NPAIR
        if not iwaited[t]:
            icopies[t].wait()
            iwaited[t] = True
        return pltpu.async_copy(
            tok_hbm.at[idx_v.at[t, pl.ds(q * _CH, _CH)]],
            toks[ch % _NTOK], gsems[ch % _NTOK])

    def start_pos(q):
        b = q % _NPOS
        return pltpu.async_copy(
            pos_hbm.at[pl.ds(p0 + q * _CH, _CH)], poss[b], psems[b])

    gathers = [None] * _NCH
    ploads = [None] * _NQ
    stores = [None] * _NCH
    gathers[0] = start_gather(0)
    gathers[1] = start_gather(1)
    ploads[0] = start_pos(0)
    for ch in range(_NCH):
        tb = ch % _NTOK
        q, t = ch // _NPAIR, ch % _NPAIR
        pb = q % _NPOS
        if ch + 2 < _NCH:
            if ch >= 2 and stores[ch - 2] is not None:
                stores[ch - 2].wait()
            gathers[ch + 2] = start_gather(ch + 2)
        if t == 0:
            if q + 1 < _NQ:
                ploads[q + 1] = start_pos(q + 1)
        gathers[ch].wait()
        if t == 0:
            ploads[q].wait()

        def row_body(r, carry):
            for k in range(_VPR):
                tv = toks[tb][r, pl.ds(k * _LANES, _LANES)]
                pv = poss[pb][r, pl.ds(k * _LANES, _LANES)]
                toks[tb][r, pl.ds(k * _LANES, _LANES)] = tv * _SCALE + pv
            return carry
        lax.fori_loop(0, _CH, row_body, 0, unroll=False)

        out_base = (2 * g + t) * _S + p0 + q * _CH
        stores[ch] = pltpu.async_copy(
            toks[tb], out_hbm.at[pl.ds(out_base, _CH)], ssems[tb])
    stores[_NCH - 2].wait()
    stores[_NCH - 1].wait()


def kernel(x, token_table, pos_table):
    xf = x.reshape(_B * _S).astype(jnp.int32)
    mesh = plsc.VectorSubcoreMesh(core_axis_name="c", subcore_axis_name="s")
    run = pl.kernel(
        _embed_kernel,
        out_type=jax.ShapeDtypeStruct((_B * _S, _D), jnp.float32),
        mesh=mesh,
        scratch_types=(
            [pltpu.VMEM((_NPAIR, _PPW), jnp.int32)]
            + [pltpu.VMEM((_CH, _D), jnp.float32)
               for _ in range(_NTOK + _NPOS)]
            + [pltpu.SemaphoreType.DMA
               for _ in range(2 * _NTOK + _NPOS + _NPAIR)]
        ),
    )
    out = run(xf, token_table, pos_table)
    return out.reshape(_B, _S, _D)


# speedup vs baseline: 2.1220x; 1.0015x over previous
import math

import jax
import jax.numpy as jnp
from jax import lax
from jax.experimental import pallas as pl
from jax.experimental.pallas import tpu as pltpu
from jax.experimental.pallas import tpu_sc as plsc

_B, _S, _D = 4, 2048, 1024
_SCALE = math.sqrt(_D)
_NW = 32
_PPW = _S // _NW
_RPW = _B * _PPW
_CH = 16
_NQ = _PPW // _CH
_NCH = _B * _NQ
_LANES = 16
_VPR = _D // _LANES
_NTOK = 4
_NPOS = 2


def _embed_kernel(x_hbm, tok_hbm, pos_hbm, out_hbm, idx_v,
                  tok0, tok1, tok2, tok3, pos0, pos1,
                  gs0, gs1, gs2, gs3, ps0, ps1, ss0, ss1, ss2, ss3,
                  is0, is1, is2, is3):
    toks = (tok0, tok1, tok2, tok3)
    poss = (pos0, pos1)
    gsems = (gs0, gs1, gs2, gs3)
    psems = (ps0, ps1)
    ssems = (ss0, ss1, ss2, ss3)
    isems = (is0, is1, is2, is3)

    c = lax.axis_index("c")
    s = lax.axis_index("s")
    wid = s * 2 + c
    p0 = wid * _PPW

    icopies = [pltpu.async_copy(
        x_hbm.at[pl.ds(b * _S + p0, _PPW)], idx_v.at[b], isems[b])
        for b in range(_B)]
    iwaited = [False] * _B

    def start_gather(ch):
        q, b = ch // _B, ch % _B
        if not iwaited[b]:
            icopies[b].wait()
            iwaited[b] = True
        return pltpu.async_copy(
            tok_hbm.at[idx_v.at[b, pl.ds(q * _CH, _CH)]],
            toks[ch % _NTOK], gsems[ch % _NTOK])

    def start_pos(q):
        return pltpu.async_copy(
            pos_hbm.at[pl.ds(p0 + q * _CH, _CH)], poss[q % _NPOS],
            psems[q % _NPOS])

    gathers = [None] * _NCH
    ploads = [None] * _NQ
    stores = [None] * _NCH
    gathers[0] = start_gather(0)
    gathers[1] = start_gather(1)
    ploads[0] = start_pos(0)
    for ch in range(_NCH):
        tb = ch % _NTOK
        q, b = ch // _B, ch % _B
        pb = q % _NPOS
        if ch + 2 < _NCH:
            if ch >= 2 and stores[ch - 2] is not None:
                stores[ch - 2].wait()
            gathers[ch + 2] = start_gather(ch + 2)
        if b == 0:
            if q + 1 < _NQ:
                ploads[q + 1] = start_pos(q + 1)
        gathers[ch].wait()
        if b == 0:
            ploads[q].wait()

        def row_body(r, carry):
            for k in range(_VPR):
                tv = toks[tb][r, pl.ds(k * _LANES, _LANES)]
                pv = poss[pb][r, pl.ds(k * _LANES, _LANES)]
                toks[tb][r, pl.ds(k * _LANES, _LANES)] = tv * _SCALE + pv
            return carry
        lax.fori_loop(0, _CH, row_body, 0, unroll=False)

        out_base = b * _S + p0 + q * _CH
        stores[ch] = pltpu.async_copy(
            toks[tb], out_hbm.at[pl.ds(out_base, _CH)], ssems[tb])
    stores[_NCH - 2].wait()
    stores[_NCH - 1].wait()


def kernel(x, token_table, pos_table):
    xf = x.reshape(_B * _S).astype(jnp.int32)
    mesh = plsc.VectorSubcoreMesh(core_axis_name="c", subcore_axis_name="s")
    run = pl.kernel(
        _embed_kernel,
        out_type=jax.ShapeDtypeStruct((_B * _S, _D), jnp.float32),
        mesh=mesh,
        scratch_types=(
            [pltpu.VMEM((_B, _PPW), jnp.int32)]
            + [pltpu.VMEM((_CH, _D), jnp.float32)
               for _ in range(_NTOK + _NPOS)]
            + [pltpu.SemaphoreType.DMA
               for _ in range(2 * _NTOK + _NPOS + _B)]
        ),
    )
    out = run(xf, token_table, pos_table)
    return out.reshape(_B, _S, _D)


# speedup vs baseline: 2.1368x; 1.0070x over previous
import math

import jax
import jax.numpy as jnp
from jax import lax
from jax.experimental import pallas as pl
from jax.experimental.pallas import tpu as pltpu
from jax.experimental.pallas import tpu_sc as plsc

_B, _S, _D = 4, 2048, 1024
_SCALE = math.sqrt(_D)
_NW = 32
_PPW = _S // _NW
_RPW = _B * _PPW
_CH = 16
_NQ = _PPW // _CH
_NCH = _B * _NQ
_LANES = 16
_VPR = _D // _LANES
_NTOK = 6
_NPOS = 1


def _embed_kernel(x_hbm, tok_hbm, pos_hbm, out_hbm, idx_v,
                  tok0, tok1, tok2, tok3, tok4, tok5, pos0,
                  gs0, gs1, gs2, gs3, gs4, gs5, ps0,
                  ss0, ss1, ss2, ss3, ss4, ss5,
                  is0, is1, is2, is3):
    toks = (tok0, tok1, tok2, tok3, tok4, tok5)
    poss = (pos0,)
    gsems = (gs0, gs1, gs2, gs3, gs4, gs5)
    psems = (ps0,)
    ssems = (ss0, ss1, ss2, ss3, ss4, ss5)
    isems = (is0, is1, is2, is3)

    c = lax.axis_index("c")
    s = lax.axis_index("s")
    wid = s * 2 + c
    p0 = wid * _PPW

    icopies = [pltpu.async_copy(
        x_hbm.at[pl.ds(b * _S + p0, _PPW)], idx_v.at[b], isems[b])
        for b in range(_B)]
    iwaited = [False] * _B

    def start_gather(ch):
        q, b = ch // _B, ch % _B
        if not iwaited[b]:
            icopies[b].wait()
            iwaited[b] = True
        return pltpu.async_copy(
            tok_hbm.at[idx_v.at[b, pl.ds(q * _CH, _CH)]],
            toks[ch % _NTOK], gsems[ch % _NTOK])

    def start_pos(q):
        return pltpu.async_copy(
            pos_hbm.at[pl.ds(p0 + q * _CH, _CH)], poss[q % _NPOS],
            psems[q % _NPOS])

    gathers = [None] * _NCH
    ploads = [None] * _NQ
    stores = [None] * _NCH
    gathers[0] = start_gather(0)
    gathers[1] = start_gather(1)
    ploads[0] = start_pos(0)
    for ch in range(0, _NCH, 2):
        q, b = ch // _B, ch % _B
        ta, tb = ch % _NTOK, (ch + 1) % _NTOK
        pb = q % _NPOS
        if ch + 2 < _NCH:
            if ch >= 4:
                stores[ch - 4].wait()
                stores[ch - 3].wait()
            gathers[ch + 2] = start_gather(ch + 2)
            gathers[ch + 3] = start_gather(ch + 3)
        gathers[ch].wait()
        gathers[ch + 1].wait()
        if b == 0:
            ploads[q].wait()

        def row_body(r, carry):
            for k in range(_VPR):
                sl = pl.ds(k * _LANES, _LANES)
                pv = poss[pb][r, sl]
                t0 = toks[ta][r, sl]
                t1 = toks[tb][r, sl]
                toks[ta][r, sl] = t0 * _SCALE + pv
                toks[tb][r, sl] = t1 * _SCALE + pv
            return carry
        lax.fori_loop(0, _CH, row_body, 0, unroll=False)

        out_base = b * _S + p0 + q * _CH
        stores[ch] = pltpu.async_copy(
            toks[ta], out_hbm.at[pl.ds(out_base, _CH)], ssems[ta])
        stores[ch + 1] = pltpu.async_copy(
            toks[tb], out_hbm.at[pl.ds(out_base + _S, _CH)], ssems[tb])
        if b == 2 and q + 1 < _NQ:
            ploads[q + 1] = start_pos(q + 1)
    for j in range(4):
        stores[_NCH - 4 + j].wait()


def kernel(x, token_table, pos_table):
    xf = x.reshape(_B * _S).astype(jnp.int32)
    mesh = plsc.VectorSubcoreMesh(core_axis_name="c", subcore_axis_name="s")
    run = pl.kernel(
        _embed_kernel,
        out_type=jax.ShapeDtypeStruct((_B * _S, _D), jnp.float32),
        mesh=mesh,
        scratch_types=(
            [pltpu.VMEM((_B, _PPW), jnp.int32)]
            + [pltpu.VMEM((_CH, _D), jnp.float32)
               for _ in range(_NTOK + _NPOS)]
            + [pltpu.SemaphoreType.DMA
               for _ in range(2 * _NTOK + _NPOS + _B)]
        ),
    )
    out = run(xf, token_table, pos_table)
    return out.reshape(_B, _S, _D)
